# Initial kernel scaffold; baseline (speedup 1.0000x reference)
import jax
import jax.numpy as jnp
from jax.experimental import pallas as pl


def kernel(x, edge_index, idx_batch, x_tab, W1_l, b1_l, W1_r, W2_l, b2_l, W2_r, Wm1, bm1, Wm2, bm2, Wm3, bm3):
    raise NotImplementedError("write your pallas kernel here")


# speedup vs baseline: 2.0492x; 2.0492x over previous
import jax
import jax.numpy as jnp
from jax import lax
from jax.experimental import pallas as pl
from jax.experimental.pallas import tpu as pltpu
from jax.experimental.pallas import tpu_sc as plsc

N = 100000
E = 1600000
B = 16384
HID = 64

NC = 2
NS = 16
NPAD = 100352
SUBN = NPAD // NS
EPAD = 1638400
ECH = EPAD // NC
ESUB = ECH // NS
KE = 2048
NCHUNK = ESUB // KE
NCHUNK2 = EPAD // NS // KE
HS = B // NC
HS2 = HS // 2
TJ = HS2 + 128
TSUB = TJ // NS
CB = 2176

_mesh = plsc.VectorSubcoreMesh(core_axis_name="c", subcore_axis_name="s")
_params = pltpu.CompilerParams(needs_layout_passes=False)


def _sc_kernel_a(src1, dst1, xp0, xp1, zn, on_h, parts,
                 a0_sp, a1_sp, cn_sp, xs0, xs1,
                 srcv, dstv, g0, g1, onev, sem):
    c = lax.axis_index("c")
    s = lax.axis_index("s")
    sl = pl.ds(s * SUBN, SUBN)
    pltpu.sync_copy(zn.at[sl], a0_sp.at[sl])
    pltpu.sync_copy(zn.at[sl], a1_sp.at[sl])
    pltpu.sync_copy(zn.at[sl], cn_sp.at[sl])
    pltpu.sync_copy(xp0.at[sl], xs0.at[sl])
    pltpu.sync_copy(xp1.at[sl], xs1.at[sl])
    pltpu.sync_copy(on_h, onev)
    plsc.subcore_barrier()

    base0 = c * ECH + s * ESUB

    def chunk(j, _):
        base = base0 + j * KE
        pltpu.sync_copy(src1.at[pl.ds(base, KE)], srcv)
        pltpu.sync_copy(dst1.at[pl.ds(base, KE)], dstv)
        pltpu.async_copy(xs0.at[srcv], g0, sem).wait()
        pltpu.async_copy(xs1.at[srcv], g1, sem).wait()
        pltpu.sync_copy(g0, a0_sp.at[dstv], add=True)
        pltpu.sync_copy(g1, a1_sp.at[dstv], add=True)
        pltpu.sync_copy(onev, cn_sp.at[dstv], add=True)
        return 0

    lax.fori_loop(0, NCHUNK, chunk, 0)
    plsc.subcore_barrier()
    off = s * SUBN
    pltpu.sync_copy(a0_sp.at[sl], parts.at[pl.ds((c * 3 + 0) * NPAD + off, SUBN)])
    pltpu.sync_copy(a1_sp.at[sl], parts.at[pl.ds((c * 3 + 1) * NPAD + off, SUBN)])
    pltpu.sync_copy(cn_sp.at[sl], parts.at[pl.ds((c * 3 + 2) * NPAD + off, SUBN)])


def _sc_kernel_b(src1, dst1, h1w, cnt1, neg1, idxb, iotb,
                 aggb1, parw, hb128, cntb,
                 slot_sp, agg2_sp,
                 srcv, dstv, slotv, csrc, cslot, rows, zbuf,
                 bidx, bval, bloc, didx, cvalf, sem):
    c = lax.axis_index("c")
    s = lax.axis_index("s")
    lo = c * HS
    sl = pl.ds(s * SUBN, SUBN)
    pltpu.sync_copy(neg1.at[sl], slot_sp.at[sl])

    def zfill(i, _):
        for k in range(8):
            zbuf[i, pl.ds(k * 16, 16)] = jnp.zeros((16,), jnp.float32)
        return 0

    lax.fori_loop(0, TSUB, zfill, 0)
    pltpu.sync_copy(zbuf, agg2_sp.at[pl.ds(s * TSUB, TSUB)])
    plsc.subcore_barrier()

    @pl.when(s == 0)
    def _():
        for m in range(8):
            pltpu.sync_copy(idxb.at[pl.ds(m * KE, KE)], srcv)
            pltpu.sync_copy(iotb.at[pl.ds(m * KE, KE)], dstv)
            pltpu.sync_copy(dstv, slot_sp.at[srcv])

    plsc.subcore_barrier()

    base0 = s * (EPAD // NS)
    lane = lax.iota(jnp.int32, 16)

    def chunk(j, _):
        base = base0 + j * KE
        pltpu.sync_copy(src1.at[pl.ds(base, KE)], srcv)
        pltpu.sync_copy(dst1.at[pl.ds(base, KE)], dstv)
        pltpu.async_copy(slot_sp.at[dstv], slotv, sem).wait()

        def prefill(i, _):
            csrc[pl.ds(i * 16, 16)] = jnp.full(
                (16,), ((N >> 11) << 12) + (N & 2047), jnp.int32)
            cslot[pl.ds(i * 16, 16)] = jnp.full((16,), HS2, jnp.int32)
            return 0
        lax.fori_loop(0, CB // 16, prefill, 0)

        def comp(i, ptr):
            slv = slotv[pl.ds(i * 16, 16)] - lo
            srv = srcv[pl.ds(i * 16, 16)]
            msk = (slv >= 0) & (slv < HS)
            mi = msk.astype(jnp.int32)
            inc = plsc.cumsum(mi)
            dest = (inc - mi) + ptr
            plsc.store_scatter(cslot, [dest], slv >> 1, mask=msk)
            hrow = (((srv >> 11) << 12) + (srv & 2047)) + ((slv & 1) << 11)
            plsc.store_scatter(csrc, [dest], hrow, mask=msk)
            return jnp.minimum(ptr + jnp.sum(mi), KE)

        ptr = lax.fori_loop(0, KE // 16, comp, 0)
        ptr = jnp.minimum(jnp.maximum(ptr, 0), KE)
        nsub = (ptr + 127) // 128

        def sub(t, _):
            pltpu.async_copy(h1w.at[csrc.at[pl.ds(t * 128, 128)]], rows, sem).wait()
            pltpu.sync_copy(rows, agg2_sp.at[cslot.at[pl.ds(t * 128, 128)]],
                            add=True)
            return 0

        lax.fori_loop(0, nsub, sub, 0)
        return 0

    lax.fori_loop(0, NCHUNK2, chunk, 0)
    plsc.subcore_barrier()

    for m in range(8):
        r = s * 8 + m
        pltpu.sync_copy(idxb.at[pl.ds(r * 128, 128)], bidx)
        pltpu.async_copy(slot_sp.at[bidx], bval, sem).wait()
        for g in range(8):
            gs = pl.ds(g * 16, 16)
            slv = bval[gs] - lo
            inr = (slv >= 0) & (slv < HS)
            bloc[gs] = jnp.where(inr, slv >> 1, HS2)
            didx[gs] = jnp.where(inr, r * 128 + g * 16 + lane, B)
            cvalf[gs] = (slv & 1).astype(jnp.float32)
        pltpu.async_copy(agg2_sp.at[bloc], rows, sem).wait()
        pltpu.async_copy(rows, aggb1.at[didx], sem).wait()
        pltpu.async_copy(cvalf, parw.at[didx], sem).wait()

    @pl.when(c == 0)
    def _():
        for m in range(8):
            r = s * 8 + m
            pltpu.sync_copy(idxb.at[pl.ds(r * 128, 128)], bidx)
            for g in range(8):
                gs = pl.ds(g * 16, 16)
                bv = bidx[gs]
                bloc[gs] = ((bv >> 11) << 12) + (bv & 2047)
            pltpu.async_copy(h1w.at[bloc], rows, sem).wait()
            pltpu.sync_copy(rows, hb128.at[pl.ds(r * 128, 128)])
            pltpu.async_copy(cnt1.at[bidx], cvalf, sem).wait()
            pltpu.sync_copy(cvalf, cntb.at[pl.ds(r * 128, 128)])


def _tc1_body(parts_ref, x_ref, w1l_ref, w1r_ref, b1_ref, h1_ref, cnt_ref):
    a0 = parts_ref[0, :] + parts_ref[3, :]
    a1 = parts_ref[1, :] + parts_ref[4, :]
    cn = parts_ref[2, :] + parts_ref[5, :]
    rc = 1.0 / jnp.maximum(cn, 1.0)
    mean1 = jnp.concatenate([(a0 * rc)[:, None], (a1 * rc)[:, None]], axis=1)
    h = (jnp.dot(mean1, w1l_ref[...])
         + jnp.dot(x_ref[...], w1r_ref[...])
         + b1_ref[0, :][None, :])
    hr = jnp.maximum(h, 0.0)
    z = jnp.zeros_like(hr)
    nr = hr.shape[0]
    h1_ref[0:nr, :] = jnp.concatenate([hr, z], axis=1)
    h1_ref[nr:2 * nr, :] = jnp.concatenate([z, hr], axis=1)
    cnt_ref[...] = cn[:, None]


def _tc2_body(aggb_ref, parw_ref, hb_ref, cntb_ref, xtab_ref,
              w2l_ref, w2r_ref, b2_ref, wm1h_ref, wm1t_ref, bm1_ref,
              wm2_ref, bm2_ref, wm3_ref, bm3_ref, out_ref):
    rc = 1.0 / jnp.maximum(cntb_ref[...], 1.0)
    sel = parw_ref[...] > 0.5
    agg = jnp.where(sel, aggb_ref[:, HID:], aggb_ref[:, :HID])
    mean2 = agg * rc
    h2 = jnp.maximum(
        jnp.dot(mean2, w2l_ref[...], preferred_element_type=jnp.float32)
        + jnp.dot(hb_ref[:, :HID], w2r_ref[...], preferred_element_type=jnp.float32)
        + b2_ref[0, :][None, :], 0.0)
    z1 = jnp.maximum(
        jnp.dot(h2, wm1h_ref[...], preferred_element_type=jnp.float32)
        + jnp.dot(xtab_ref[...], wm1t_ref[...], preferred_element_type=jnp.float32)
        + bm1_ref[0, :][None, :], 0.0)
    z2 = jnp.maximum(
        jnp.dot(z1, wm2_ref[...], preferred_element_type=jnp.float32)
        + bm2_ref[0, :][None, :], 0.0)
    out_ref[...] = (jnp.sum(z2 * wm3_ref[0, :][None, :], axis=1)
                    + bm3_ref[0, 0])[:, None]


@jax.jit
def kernel(x, edge_index, idx_batch, x_tab, W1_l, b1_l, W1_r, W2_l, b2_l, W2_r,
           Wm1, bm1, Wm2, bm2, Wm3, bm3):
    f32 = jnp.float32
    i32 = jnp.int32

    srcp = jnp.full((EPAD,), N, i32).at[:E].set(edge_index[0])
    dstp = jnp.full((EPAD,), N, i32).at[:E].set(edge_index[1])
    xp0 = jnp.zeros((NPAD,), f32).at[:N].set(x[:, 0])
    xp1 = jnp.zeros((NPAD,), f32).at[:N].set(x[:, 1])
    zn = jnp.zeros((NPAD,), f32)
    neg1 = jnp.full((NPAD,), -1, i32)
    iotb = jnp.arange(B, dtype=i32)
    xp = jnp.zeros((NPAD, 2), f32).at[:N].set(x)

    parts = pl.kernel(
        _sc_kernel_a,
        out_type=jax.ShapeDtypeStruct((NC * 3 * NPAD,), f32),
        mesh=_mesh,
        compiler_params=_params,
        scratch_types=[
            pltpu.VMEM_SHARED((NPAD,), f32),
            pltpu.VMEM_SHARED((NPAD,), f32),
            pltpu.VMEM_SHARED((NPAD,), f32),
            pltpu.VMEM_SHARED((NPAD,), f32),
            pltpu.VMEM_SHARED((NPAD,), f32),
            pltpu.VMEM((KE,), i32),
            pltpu.VMEM((KE,), i32),
            pltpu.VMEM((KE,), f32),
            pltpu.VMEM((KE,), f32),
            pltpu.VMEM((KE,), f32),
            pltpu.SemaphoreType.DMA,
        ],
    )(srcp, dstp, xp0, xp1, zn, jnp.ones((KE,), f32))

    R1 = 2048
    G1 = NPAD // R1

    h1w, cnt2 = pl.pallas_call(
        _tc1_body,
        grid=(G1,),
        in_specs=[
            pl.BlockSpec((NC * 3, R1), lambda i: (0, i)),
            pl.BlockSpec((R1, 2), lambda i: (i, 0)),
            pl.BlockSpec((2, HID), lambda i: (0, 0)),
            pl.BlockSpec((2, HID), lambda i: (0, 0)),
            pl.BlockSpec((1, HID), lambda i: (0, 0)),
        ],
        out_specs=[
            pl.BlockSpec((2 * R1, 2 * HID), lambda i: (i, 0)),
            pl.BlockSpec((R1, 1), lambda i: (i, 0)),
        ],
        out_shape=[
            jax.ShapeDtypeStruct((2 * NPAD, 2 * HID), f32),
            jax.ShapeDtypeStruct((NPAD, 1), f32),
        ],
    )(parts.reshape(NC * 3, NPAD),
      xp, W1_l.T, W1_r.T, b1_l.reshape(1, HID))
    cnt1 = cnt2.reshape(NPAD)

    aggb1, parw, hb128, cntb = pl.kernel(
        _sc_kernel_b,
        out_type=(
            jax.ShapeDtypeStruct((B + 128, 2 * HID), f32),
            jax.ShapeDtypeStruct((B + 128,), f32),
            jax.ShapeDtypeStruct((B, 2 * HID), f32),
            jax.ShapeDtypeStruct((B,), f32),
        ),
        mesh=_mesh,
        compiler_params=_params,
        scratch_types=[
            pltpu.VMEM_SHARED((NPAD,), i32),
            pltpu.VMEM_SHARED((TJ, 2 * HID), f32),
            pltpu.VMEM((KE,), i32),
            pltpu.VMEM((KE,), i32),
            pltpu.VMEM((KE,), i32),
            pltpu.VMEM((CB,), i32),
            pltpu.VMEM((CB,), i32),
            pltpu.VMEM((128, 2 * HID), f32),
            pltpu.VMEM((TSUB, 2 * HID), f32),
            pltpu.VMEM((128,), i32),
            pltpu.VMEM((128,), i32),
            pltpu.VMEM((128,), i32),
            pltpu.VMEM((128,), i32),
            pltpu.VMEM((128,), f32),
            pltpu.SemaphoreType.DMA,
        ],
    )(srcp, dstp, h1w, cnt1, neg1, idx_batch, iotb)

    Bb = 2048
    G2 = B // Bb
    out = pl.pallas_call(
        _tc2_body,
        grid=(G2,),
        in_specs=[
            pl.BlockSpec((Bb, 2 * HID), lambda i: (i, 0)),
            pl.BlockSpec((Bb, 1), lambda i: (i, 0)),
            pl.BlockSpec((Bb, 2 * HID), lambda i: (i, 0)),
            pl.BlockSpec((Bb, 1), lambda i: (i, 0)),
            pl.BlockSpec((Bb, 4), lambda i: (i, 0)),
            pl.BlockSpec((HID, HID), lambda i: (0, 0)),
            pl.BlockSpec((HID, HID), lambda i: (0, 0)),
            pl.BlockSpec((1, HID), lambda i: (0, 0)),
            pl.BlockSpec((HID, 64), lambda i: (0, 0)),
            pl.BlockSpec((4, 64), lambda i: (0, 0)),
            pl.BlockSpec((1, 64), lambda i: (0, 0)),
            pl.BlockSpec((64, 32), lambda i: (0, 0)),
            pl.BlockSpec((1, 32), lambda i: (0, 0)),
            pl.BlockSpec((1, 32), lambda i: (0, 0)),
            pl.BlockSpec((1, 1), lambda i: (0, 0)),
        ],
        out_specs=pl.BlockSpec((Bb, 1), lambda i: (i, 0)),
        out_shape=jax.ShapeDtypeStruct((B, 1), f32),
    )(aggb1, parw[:B].reshape(B, 1), hb128, cntb.reshape(B, 1), x_tab,
      W2_l.T, W2_r.T, b2_l.reshape(1, HID),
      Wm1[:, :HID].T, Wm1[:, HID:].T, bm1.reshape(1, 64),
      Wm2.T, bm2.reshape(1, 32), Wm3.reshape(1, 32), bm3.reshape(1, 1))
    return out.reshape(B)


# speedup vs baseline: 5.1636x; 2.5198x over previous
import jax
import jax.numpy as jnp
from jax import lax
from jax.experimental import pallas as pl
from jax.experimental.pallas import tpu as pltpu
from jax.experimental.pallas import tpu_sc as plsc

N = 100000
E = 1600000
B = 16384
HID = 64

NC = 2
NS = 16
NPAD = 100352
SUBN = NPAD // NS
EPAD = 1638400
ECH = EPAD // NC
ESUB = ECH // NS
KE = 2048
NCHUNK = ESUB // KE
NCHUNK2 = EPAD // NS // KE
HS = B // NC
HS2 = HS // 2
TJ = HS2 + 128
TSUB = TJ // NS
CB = 2176

_mesh = plsc.VectorSubcoreMesh(core_axis_name="c", subcore_axis_name="s")
_params = pltpu.CompilerParams(needs_layout_passes=False)


def _sc_kernel_a(src1, dst1, xp0, xp1, zn, on_h, parts,
                 a0_sp, a1_sp, cn_sp, xs0, xs1,
                 srcv, dstv, g0, g1, onev, sem):
    c = lax.axis_index("c")
    s = lax.axis_index("s")
    sl = pl.ds(s * SUBN, SUBN)
    pltpu.sync_copy(zn.at[sl], a0_sp.at[sl])
    pltpu.sync_copy(zn.at[sl], a1_sp.at[sl])
    pltpu.sync_copy(zn.at[sl], cn_sp.at[sl])
    pltpu.sync_copy(xp0.at[sl], xs0.at[sl])
    pltpu.sync_copy(xp1.at[sl], xs1.at[sl])
    pltpu.sync_copy(on_h, onev)
    plsc.subcore_barrier()

    base0 = c * ECH + s * ESUB

    def chunk(j, _):
        base = base0 + j * KE
        pltpu.sync_copy(src1.at[pl.ds(base, KE)], srcv)
        pltpu.sync_copy(dst1.at[pl.ds(base, KE)], dstv)
        pltpu.async_copy(xs0.at[srcv], g0, sem).wait()
        pltpu.async_copy(xs1.at[srcv], g1, sem).wait()
        pltpu.sync_copy(g0, a0_sp.at[dstv], add=True)
        pltpu.sync_copy(g1, a1_sp.at[dstv], add=True)
        pltpu.sync_copy(onev, cn_sp.at[dstv], add=True)
        return 0

    lax.fori_loop(0, NCHUNK, chunk, 0)
    plsc.subcore_barrier()
    off = s * SUBN
    pltpu.sync_copy(a0_sp.at[sl], parts.at[pl.ds((c * 3 + 0) * NPAD + off, SUBN)])
    pltpu.sync_copy(a1_sp.at[sl], parts.at[pl.ds((c * 3 + 1) * NPAD + off, SUBN)])
    pltpu.sync_copy(cn_sp.at[sl], parts.at[pl.ds((c * 3 + 2) * NPAD + off, SUBN)])


def _sc_kernel_b(src1, dst1, h1w, cnt1, neg1, idxb, iotb,
                 aggb1, parw, hb128, cntb,
                 slot_sp, agg2_sp,
                 srcv, dstv, slotv, csrc, cslot, rows, zbuf,
                 bidx, bval, bloc, didx, cvalf, sem):
    c = lax.axis_index("c")
    s = lax.axis_index("s")
    lo = c * HS
    sl = pl.ds(s * SUBN, SUBN)
    pltpu.sync_copy(neg1.at[sl], slot_sp.at[sl])

    def zfill(i, _):
        for k in range(8):
            zbuf[i, pl.ds(k * 16, 16)] = jnp.zeros((16,), jnp.float32)
        return 0

    lax.fori_loop(0, TSUB, zfill, 0)
    pltpu.sync_copy(zbuf, agg2_sp.at[pl.ds(s * TSUB, TSUB)])
    plsc.subcore_barrier()

    @pl.when(s == 0)
    def _():
        for m in range(8):
            pltpu.sync_copy(idxb.at[pl.ds(m * KE, KE)], srcv)
            pltpu.sync_copy(iotb.at[pl.ds(m * KE, KE)], dstv)
            pltpu.sync_copy(dstv, slot_sp.at[srcv])

    plsc.subcore_barrier()

    base0 = s * (EPAD // NS)
    lane = lax.iota(jnp.int32, 16)

    def chunk(j, _):
        base = base0 + j * KE
        pltpu.sync_copy(src1.at[pl.ds(base, KE)], srcv)
        pltpu.sync_copy(dst1.at[pl.ds(base, KE)], dstv)
        pltpu.async_copy(slot_sp.at[dstv], slotv, sem).wait()

        def prefill(i, _):
            pos = i * 16 + lane
            csrc[pl.ds(i * 16, 16)] = pos & 2047
            cslot[pl.ds(i * 16, 16)] = HS2 + (pos & 127)
            return 0
        lax.fori_loop(0, CB // 16, prefill, 0)

        def comp(i, ptr):
            slv = slotv[pl.ds(i * 16, 16)] - lo
            srv = srcv[pl.ds(i * 16, 16)]
            msk = (slv >= 0) & (slv < HS)
            mi = msk.astype(jnp.int32)
            inc = plsc.cumsum(mi)
            dest = (inc - mi) + ptr
            plsc.store_scatter(cslot, [dest], slv >> 1, mask=msk)
            hrow = (((srv >> 11) << 12) + (srv & 2047)) + ((slv & 1) << 11)
            plsc.store_scatter(csrc, [dest], hrow, mask=msk)
            return jnp.minimum(ptr + jnp.sum(mi), KE)

        ptr = lax.fori_loop(0, KE // 16, comp, 0)
        ptr = jnp.minimum(jnp.maximum(ptr, 0), KE)
        nsub = (ptr + 127) // 128

        def sub(t, _):
            pltpu.async_copy(h1w.at[csrc.at[pl.ds(t * 128, 128)]], rows, sem).wait()
            pltpu.sync_copy(rows, agg2_sp.at[cslot.at[pl.ds(t * 128, 128)]],
                            add=True)
            return 0

        lax.fori_loop(0, nsub, sub, 0)
        return 0

    lax.fori_loop(0, NCHUNK2, chunk, 0)
    plsc.subcore_barrier()

    for m in range(8):
        r = s * 8 + m
        pltpu.sync_copy(idxb.at[pl.ds(r * 128, 128)], bidx)
        pltpu.async_copy(slot_sp.at[bidx], bval, sem).wait()
        for g in range(8):
            gs = pl.ds(g * 16, 16)
            slv = bval[gs] - lo
            inr = (slv >= 0) & (slv < HS)
            bloc[gs] = jnp.where(inr, slv >> 1, HS2)
            didx[gs] = jnp.where(inr, r * 128 + g * 16 + lane, B)
            cvalf[gs] = (slv & 1).astype(jnp.float32)
        pltpu.async_copy(agg2_sp.at[bloc], rows, sem).wait()
        pltpu.async_copy(rows, aggb1.at[didx], sem).wait()
        pltpu.async_copy(cvalf, parw.at[didx], sem).wait()

    @pl.when(c == 0)
    def _():
        for m in range(8):
            r = s * 8 + m
            pltpu.sync_copy(idxb.at[pl.ds(r * 128, 128)], bidx)
            for g in range(8):
                gs = pl.ds(g * 16, 16)
                bv = bidx[gs]
                bloc[gs] = ((bv >> 11) << 12) + (bv & 2047)
            pltpu.async_copy(h1w.at[bloc], rows, sem).wait()
            pltpu.sync_copy(rows, hb128.at[pl.ds(r * 128, 128)])
            pltpu.async_copy(cnt1.at[bidx], cvalf, sem).wait()
            pltpu.sync_copy(cvalf, cntb.at[pl.ds(r * 128, 128)])


def _tc1_body(parts_ref, x_ref, w1l_ref, w1r_ref, b1_ref, h1_ref, cnt_ref):
    a0 = parts_ref[0, :] + parts_ref[3, :]
    a1 = parts_ref[1, :] + parts_ref[4, :]
    cn = parts_ref[2, :] + parts_ref[5, :]
    rc = 1.0 / jnp.maximum(cn, 1.0)
    mean1 = jnp.concatenate([(a0 * rc)[:, None], (a1 * rc)[:, None]], axis=1)
    h = (jnp.dot(mean1, w1l_ref[...])
         + jnp.dot(x_ref[...], w1r_ref[...])
         + b1_ref[0, :][None, :])
    hr = jnp.maximum(h, 0.0)
    z = jnp.zeros_like(hr)
    nr = hr.shape[0]
    h1_ref[0:nr, :] = jnp.concatenate([hr, z], axis=1)
    h1_ref[nr:2 * nr, :] = jnp.concatenate([z, hr], axis=1)
    cnt_ref[...] = cn[:, None]


def _tc2_body(aggb_ref, parw_ref, hb_ref, cntb_ref, xtab_ref,
              w2l_ref, w2r_ref, b2_ref, wm1h_ref, wm1t_ref, bm1_ref,
              wm2_ref, bm2_ref, wm3_ref, bm3_ref, out_ref):
    rc = 1.0 / jnp.maximum(cntb_ref[...], 1.0)
    sel = parw_ref[...] > 0.5
    agg = jnp.where(sel, aggb_ref[:, HID:], aggb_ref[:, :HID])
    mean2 = agg * rc
    h2 = jnp.maximum(
        jnp.dot(mean2, w2l_ref[...], preferred_element_type=jnp.float32)
        + jnp.dot(hb_ref[:, :HID], w2r_ref[...], preferred_element_type=jnp.float32)
        + b2_ref[0, :][None, :], 0.0)
    z1 = jnp.maximum(
        jnp.dot(h2, wm1h_ref[...], preferred_element_type=jnp.float32)
        + jnp.dot(xtab_ref[...], wm1t_ref[...], preferred_element_type=jnp.float32)
        + bm1_ref[0, :][None, :], 0.0)
    z2 = jnp.maximum(
        jnp.dot(z1, wm2_ref[...], preferred_element_type=jnp.float32)
        + bm2_ref[0, :][None, :], 0.0)
    out_ref[...] = (jnp.sum(z2 * wm3_ref[0, :][None, :], axis=1)
                    + bm3_ref[0, 0])[:, None]


@jax.jit
def kernel(x, edge_index, idx_batch, x_tab, W1_l, b1_l, W1_r, W2_l, b2_l, W2_r,
           Wm1, bm1, Wm2, bm2, Wm3, bm3):
    f32 = jnp.float32
    i32 = jnp.int32

    srcp = jnp.full((EPAD,), N, i32).at[:E].set(edge_index[0])
    dstp = jnp.full((EPAD,), N, i32).at[:E].set(edge_index[1])
    xp0 = jnp.zeros((NPAD,), f32).at[:N].set(x[:, 0])
    xp1 = jnp.zeros((NPAD,), f32).at[:N].set(x[:, 1])
    zn = jnp.zeros((NPAD,), f32)
    neg1 = jnp.full((NPAD,), -1, i32)
    iotb = jnp.arange(B, dtype=i32)
    xp = jnp.zeros((NPAD, 2), f32).at[:N].set(x)

    parts = pl.kernel(
        _sc_kernel_a,
        out_type=jax.ShapeDtypeStruct((NC * 3 * NPAD,), f32),
        mesh=_mesh,
        compiler_params=_params,
        scratch_types=[
            pltpu.VMEM_SHARED((NPAD,), f32),
            pltpu.VMEM_SHARED((NPAD,), f32),
            pltpu.VMEM_SHARED((NPAD,), f32),
            pltpu.VMEM_SHARED((NPAD,), f32),
            pltpu.VMEM_SHARED((NPAD,), f32),
            pltpu.VMEM((KE,), i32),
            pltpu.VMEM((KE,), i32),
            pltpu.VMEM((KE,), f32),
            pltpu.VMEM((KE,), f32),
            pltpu.VMEM((KE,), f32),
            pltpu.SemaphoreType.DMA,
        ],
    )(srcp, dstp, xp0, xp1, zn, jnp.ones((KE,), f32))

    R1 = 2048
    G1 = NPAD // R1

    h1w, cnt2 = pl.pallas_call(
        _tc1_body,
        grid=(G1,),
        in_specs=[
            pl.BlockSpec((NC * 3, R1), lambda i: (0, i)),
            pl.BlockSpec((R1, 2), lambda i: (i, 0)),
            pl.BlockSpec((2, HID), lambda i: (0, 0)),
            pl.BlockSpec((2, HID), lambda i: (0, 0)),
            pl.BlockSpec((1, HID), lambda i: (0, 0)),
        ],
        out_specs=[
            pl.BlockSpec((2 * R1, 2 * HID), lambda i: (i, 0)),
            pl.BlockSpec((R1, 1), lambda i: (i, 0)),
        ],
        out_shape=[
            jax.ShapeDtypeStruct((2 * NPAD, 2 * HID), f32),
            jax.ShapeDtypeStruct((NPAD, 1), f32),
        ],
    )(parts.reshape(NC * 3, NPAD),
      xp, W1_l.T, W1_r.T, b1_l.reshape(1, HID))
    cnt1 = cnt2.reshape(NPAD)

    aggb1, parw, hb128, cntb = pl.kernel(
        _sc_kernel_b,
        out_type=(
            jax.ShapeDtypeStruct((B + 128, 2 * HID), f32),
            jax.ShapeDtypeStruct((B + 128,), f32),
            jax.ShapeDtypeStruct((B, 2 * HID), f32),
            jax.ShapeDtypeStruct((B,), f32),
        ),
        mesh=_mesh,
        compiler_params=_params,
        scratch_types=[
            pltpu.VMEM_SHARED((NPAD,), i32),
            pltpu.VMEM_SHARED((TJ, 2 * HID), f32),
            pltpu.VMEM((KE,), i32),
            pltpu.VMEM((KE,), i32),
            pltpu.VMEM((KE,), i32),
            pltpu.VMEM((CB,), i32),
            pltpu.VMEM((CB,), i32),
            pltpu.VMEM((128, 2 * HID), f32),
            pltpu.VMEM((TSUB, 2 * HID), f32),
            pltpu.VMEM((128,), i32),
            pltpu.VMEM((128,), i32),
            pltpu.VMEM((128,), i32),
            pltpu.VMEM((128,), i32),
            pltpu.VMEM((128,), f32),
            pltpu.SemaphoreType.DMA,
        ],
    )(srcp, dstp, h1w, cnt1, neg1, idx_batch, iotb)

    Bb = 2048
    G2 = B // Bb
    out = pl.pallas_call(
        _tc2_body,
        grid=(G2,),
        in_specs=[
            pl.BlockSpec((Bb, 2 * HID), lambda i: (i, 0)),
            pl.BlockSpec((Bb, 1), lambda i: (i, 0)),
            pl.BlockSpec((Bb, 2 * HID), lambda i: (i, 0)),
            pl.BlockSpec((Bb, 1), lambda i: (i, 0)),
            pl.BlockSpec((Bb, 4), lambda i: (i, 0)),
            pl.BlockSpec((HID, HID), lambda i: (0, 0)),
            pl.BlockSpec((HID, HID), lambda i: (0, 0)),
            pl.BlockSpec((1, HID), lambda i: (0, 0)),
            pl.BlockSpec((HID, 64), lambda i: (0, 0)),
            pl.BlockSpec((4, 64), lambda i: (0, 0)),
            pl.BlockSpec((1, 64), lambda i: (0, 0)),
            pl.BlockSpec((64, 32), lambda i: (0, 0)),
            pl.BlockSpec((1, 32), lambda i: (0, 0)),
            pl.BlockSpec((1, 32), lambda i: (0, 0)),
            pl.BlockSpec((1, 1), lambda i: (0, 0)),
        ],
        out_specs=pl.BlockSpec((Bb, 1), lambda i: (i, 0)),
        out_shape=jax.ShapeDtypeStruct((B, 1), f32),
    )(aggb1, parw[:B].reshape(B, 1), hb128, cntb.reshape(B, 1), x_tab,
      W2_l.T, W2_r.T, b2_l.reshape(1, HID),
      Wm1[:, :HID].T, Wm1[:, HID:].T, bm1.reshape(1, 64),
      Wm2.T, bm2.reshape(1, 32), Wm3.reshape(1, 32), bm3.reshape(1, 1))
    return out.reshape(B)
